# baseline (speedup 1.0000x reference)
import functools

import jax
import jax.numpy as jnp
from jax import lax
from jax.experimental import pallas as pl
from jax.experimental.pallas import tpu as pltpu
from jax.experimental.pallas import tpu_sc as plsc

D_MODEL = 64
SCALE = float(D_MODEL) ** 0.5

_V = 1000000
_B = 16384
_S = 50
_NC = 2
_NS = 16
_NW = _NC * _NS

_CP = pltpu.CompilerParams(use_tc_tiling_on_sc=True, needs_layout_passes=False)
_MESH = dict(core_axis_name="c", subcore_axis_name="s")


_NBLK = _V // 128
_BLK_PER_W = _NBLK // _NW
_BLK_REM = _NBLK % _NW


def _reformat_kernel(tt_hbm, tail_hbm, t2_hbm, in_v, out_v, sem):
    wid = lax.axis_index("s") * _NC + lax.axis_index("c")
    base = wid * _BLK_PER_W + jnp.minimum(wid, _BLK_REM)
    nblk = _BLK_PER_W + jnp.where(wid < _BLK_REM, 1, 0)

    iota = jax.lax.iota(jnp.int32, 16)
    rowv = [(iota + 16 * g) % 64 for g in range(8)]
    colb = [jnp.full((16,), g // 4, jnp.int32) for g in range(8)]

    def blk_body(m, _):
        t = base + m
        pltpu.sync_copy(tt_hbm.at[:, pl.ds(t * 128, 128)], in_v)

        def p_body(p, _):
            for g in range(8):
                vals = plsc.load_gather(in_v, [rowv[g], colb[g] + 2 * p])
                out_v[p, pl.ds(16 * g, 16)] = vals * SCALE
            return 0

        lax.fori_loop(0, 64, p_body, 0)
        pltpu.sync_copy(out_v, t2_hbm.at[pl.ds(t * 64, 64), :])
        return 0

    lax.fori_loop(0, nblk, blk_body, 0)

    @pl.when(wid == _NW - 1)
    def _copy_tail():
        pltpu.sync_copy(tail_hbm, in_v.at[pl.ds(0, 32), :])
        pltpu.sync_copy(in_v.at[pl.ds(0, 32), :],
                        t2_hbm.at[pl.ds(_NBLK * 64, 32), :])


@jax.jit
def _reformat(tt, tail):
    fn = functools.partial(
        pl.kernel,
        mesh=plsc.VectorSubcoreMesh(**_MESH),
        out_type=jax.ShapeDtypeStruct((_V // 2, 128), jnp.float32),
        scratch_types=[
            pltpu.VMEM((64, 128), jnp.float32),
            pltpu.VMEM((64, 128), jnp.float32),
            pltpu.SemaphoreType.DMA,
        ],
        compiler_params=_CP,
    )(_reformat_kernel)
    return fn(tt, tail)


_JT = _B // 128
_UNITS = _S * _JT
_PER_W = _UNITS // _NW


def _gather_kernel(xt_hbm, t2_hbm, out_hbm, idx_v, idx2_v, rows_v, out_s, sem):
    wid = lax.axis_index("s") * _NC + lax.axis_index("c")

    jv = [jax.lax.iota(jnp.int32, 16) + 16 * g for g in range(8)]

    def unit_body(m, _):
        u = wid * _PER_W + m
        s = u // _JT
        jt = u % _JT

        pltpu.sync_copy(xt_hbm.at[s, pl.ds(jt * 128, 128)], idx_v)

        for g in range(8):
            sl = pl.ds(16 * g, 16)
            idx2_v[sl] = lax.shift_right_logical(idx_v[sl], 1)
        pltpu.async_copy(t2_hbm.at[idx2_v], rows_v, sem).wait()

        ov = [(idx_v[pl.ds(16 * g, 16)] & 1) * 64 for g in range(8)]

        def col_body(c, _):
            for g in range(8):
                vals = plsc.load_gather(rows_v, [jv[g], ov[g] + c])
                out_s[c, pl.ds(16 * g, 16)] = vals
            return 0

        lax.fori_loop(0, D_MODEL, col_body, 0)

        pltpu.sync_copy(out_s, out_hbm.at[s, :, pl.ds(jt * 128, 128)])
        return 0

    lax.fori_loop(0, _PER_W, unit_body, 0)


@jax.jit
def _gather(xt, t2):
    fn = functools.partial(
        pl.kernel,
        mesh=plsc.VectorSubcoreMesh(**_MESH),
        out_type=jax.ShapeDtypeStruct((_S, D_MODEL, _B), jnp.float32),
        scratch_types=[
            pltpu.VMEM((128,), jnp.int32),
            pltpu.VMEM((128,), jnp.int32),
            pltpu.VMEM((128, 128), jnp.float32),
            pltpu.VMEM((D_MODEL, 128), jnp.float32),
            pltpu.SemaphoreType.DMA,
        ],
        compiler_params=_CP,
    )(_gather_kernel)
    return fn(xt, t2)


def kernel(x, table):
    xt = jnp.transpose(x)
    tt = jnp.transpose(table)
    tail = jnp.reshape(lax.slice(table, (_NBLK * 128, 0), (_V, D_MODEL)),
                       (32, 128)) * SCALE
    t2 = _reformat(tt, tail)
    out_p = _gather(xt, t2)
    return jnp.transpose(out_p, (2, 0, 1))


# speedup vs baseline: 2.2519x; 2.2519x over previous
import functools

import jax
import jax.numpy as jnp
from jax import lax
from jax.experimental import pallas as pl
from jax.experimental.pallas import tpu as pltpu
from jax.experimental.pallas import tpu_sc as plsc

D_MODEL = 64
SCALE = float(D_MODEL) ** 0.5

_V = 1000000
_B = 16384
_S = 50
_NC = 2
_NS = 16
_NW = _NC * _NS

_CP = pltpu.CompilerParams(use_tc_tiling_on_sc=True, needs_layout_passes=False)
_MESH = dict(core_axis_name="c", subcore_axis_name="s")

_IOTA16 = lambda: jax.lax.iota(jnp.int32, 16)


_NBLK = _V // 128
_GRP = 4
_NGRP = -(-(_NBLK) // (_NW * _GRP))


def _reformat_kernel(tt_hbm, tail_hbm, t2_hbm,
                     in0, in1, out0, out1, si0, si1, so0, so1):
    wid = lax.axis_index("s") * _NC + lax.axis_index("c")
    base = wid * (_NGRP * _GRP)

    iota = _IOTA16()
    rowv = [(iota + 16 * g) % 64 for g in range(8)]
    colb = [jnp.full((16,), g // 4, jnp.int32) for g in range(8)]
    ins, outs, sis, sos = [in0, in1], [out0, out1], [si0, si1], [so0, so1]

    def transpose_block(in_b, out_b):
        @plsc.parallel_loop(0, 64, step=1, unroll=4)
        def _(p):
            for g in range(8):
                vals = plsc.load_gather(in_b, [rowv[g], colb[g] + 2 * p])
                out_b[p, pl.ds(16 * g, 16)] = vals * SCALE

    def grp_body(k, _):
        t = [jnp.minimum(base + _GRP * k + i, _NBLK - 1) for i in range(_GRP)]
        h_in = [pltpu.async_copy(tt_hbm.at[:, pl.ds(t[i] * 128, 128)],
                                 ins[i], sis[i]) for i in range(2)]
        h_out = [None, None]
        for i in range(_GRP):
            b = i % 2
            h_in[b].wait()
            if h_out[b] is not None:
                h_out[b].wait()
            transpose_block(ins[b], outs[b])
            h_out[b] = pltpu.async_copy(
                outs[b], t2_hbm.at[pl.ds(t[i] * 64, 64), :], sos[b])
            if i + 2 < _GRP:
                h_in[b] = pltpu.async_copy(
                    tt_hbm.at[:, pl.ds(t[i + 2] * 128, 128)], ins[b], sis[b])
        h_out[0].wait()
        h_out[1].wait()
        return 0

    lax.fori_loop(0, _NGRP, grp_body, 0)

    @pl.when(wid == _NW - 1)
    def _copy_tail():
        pltpu.sync_copy(tail_hbm, in0.at[pl.ds(0, 32), :])
        pltpu.sync_copy(in0.at[pl.ds(0, 32), :],
                        t2_hbm.at[pl.ds(_NBLK * 64, 32), :])


@jax.jit
def _reformat(tt, tail):
    fn = functools.partial(
        pl.kernel,
        mesh=plsc.VectorSubcoreMesh(**_MESH),
        out_type=jax.ShapeDtypeStruct((_V // 2, 128), jnp.float32),
        scratch_types=[
            pltpu.VMEM((64, 128), jnp.float32),
            pltpu.VMEM((64, 128), jnp.float32),
            pltpu.VMEM((64, 128), jnp.float32),
            pltpu.VMEM((64, 128), jnp.float32),
            pltpu.SemaphoreType.DMA,
            pltpu.SemaphoreType.DMA,
            pltpu.SemaphoreType.DMA,
            pltpu.SemaphoreType.DMA,
        ],
        compiler_params=_CP,
    )(_reformat_kernel)
    return fn(tt, tail)


_JT = _B // 128
_CHU = 16
_NCHUNK = _S * (_JT // _CHU)
_CH_PER_W = -(-_NCHUNK // _NW)


def _gather_kernel(xt_hbm, t2_hbm, out_hbm,
                   idxb, ix0, ix1, rows0, rows1, os0, os1,
                   sg0, sg1, so0, so1):
    wid = lax.axis_index("s") * _NC + lax.axis_index("c")

    iota = _IOTA16()
    jv = [iota + 16 * g for g in range(8)]
    ixs, rows, oss = [ix0, ix1], [rows0, rows1], [os0, os1]
    sgs, sos = [sg0, sg1], [so0, so1]

    def prep(u, b):
        for g in range(8):
            ixs[b][pl.ds(16 * g, 16)] = lax.shift_right_logical(
                idxb[pl.ds(u * 128 + 16 * g, 16)], 1)

    def extract(u, b):
        ov = [(idxb[pl.ds(u * 128 + 16 * g, 16)] & 1) * 64 for g in range(8)]
        rb, ob = rows[b], oss[b]

        @plsc.parallel_loop(0, D_MODEL, step=1, unroll=4)
        def _(c):
            for g in range(8):
                vals = plsc.load_gather(rb, [jv[g], ov[g] + c])
                ob[c, pl.ds(16 * g, 16)] = vals

    def chunk_body(m, _):
        cid = jnp.minimum(wid * _CH_PER_W + m, _NCHUNK - 1)
        s = cid // (_JT // _CHU)
        jt0 = (cid % (_JT // _CHU)) * _CHU

        pltpu.sync_copy(xt_hbm.at[s, pl.ds(jt0 * 128, _CHU * 128)], idxb)

        prep(0, 0)
        h_g = [pltpu.async_copy(t2_hbm.at[ix0], rows0, sg0), None]
        h_o = [None, None]
        for u in range(_CHU):
            b = u % 2
            if u + 1 < _CHU:
                prep(u + 1, 1 - b)
                h_g[1 - b] = pltpu.async_copy(
                    t2_hbm.at[ixs[1 - b]], rows[1 - b], sgs[1 - b])
            h_g[b].wait()
            if h_o[b] is not None:
                h_o[b].wait()
            extract(u, b)
            h_o[b] = pltpu.async_copy(
                oss[b], out_hbm.at[s, :, pl.ds((jt0 + u) * 128, 128)], sos[b])
        h_o[0].wait()
        h_o[1].wait()
        return 0

    lax.fori_loop(0, _CH_PER_W, chunk_body, 0)


@jax.jit
def _gather(xt, t2):
    fn = functools.partial(
        pl.kernel,
        mesh=plsc.VectorSubcoreMesh(**_MESH),
        out_type=jax.ShapeDtypeStruct((_S, D_MODEL, _B), jnp.float32),
        scratch_types=[
            pltpu.VMEM((_CHU * 128,), jnp.int32),
            pltpu.VMEM((128,), jnp.int32),
            pltpu.VMEM((128,), jnp.int32),
            pltpu.VMEM((128, 128), jnp.float32),
            pltpu.VMEM((128, 128), jnp.float32),
            pltpu.VMEM((D_MODEL, 128), jnp.float32),
            pltpu.VMEM((D_MODEL, 128), jnp.float32),
            pltpu.SemaphoreType.DMA,
            pltpu.SemaphoreType.DMA,
            pltpu.SemaphoreType.DMA,
            pltpu.SemaphoreType.DMA,
        ],
        compiler_params=_CP,
    )(_gather_kernel)
    return fn(xt, t2)


def kernel(x, table):
    xt = jnp.transpose(x)
    tt = jnp.transpose(table)
    tail = jnp.reshape(lax.slice(table, (_NBLK * 128, 0), (_V, D_MODEL)),
                       (32, 128)) * SCALE
    t2 = _reformat(tt, tail)
    out_p = _gather(xt, t2)
    return jnp.transpose(out_p, (2, 0, 1))
